# baseline (speedup 1.0000x reference)
import functools

import jax
import jax.numpy as jnp
from jax import lax
from jax.experimental import pallas as pl
from jax.experimental.pallas import tpu as pltpu
from jax.experimental.pallas import tpu_sc as plsc

_B, _N, _D = 16, 2048, 3
_TILE = 256
_BIG = 1e30
_SC_L = 16


def _chamfer_tc_body(x_ref, yt_ref, inp_ref, outp_ref, acc_ref):
    x = x_ref[0]
    in_pid = inp_ref[0]
    out_pid = outp_ref[0]

    in_mask_c = in_pid != 0
    out_mask_r = out_pid != 0

    n_in = jnp.sum(in_mask_c.astype(jnp.float32))
    n_out = jnp.sum(out_mask_r.astype(jnp.float32))

    x0c = x[:, 0:1]
    x1c = x[:, 1:2]
    x2c = x[:, 2:3]
    xn2f = x0c * x0c + x1c * x1c + x2c * x2c
    x_norm = jnp.sqrt(xn2f)
    x_norm_sum = jnp.sum(jnp.where(in_mask_c, x_norm, 0.0))

    xb = x.astype(jnp.bfloat16)
    ybt = yt_ref[0].astype(jnp.bfloat16)
    xbf0 = xb[:, 0:1].astype(jnp.float32)
    xbf1 = xb[:, 1:2].astype(jnp.float32)
    xbf2 = xb[:, 2:3].astype(jnp.float32)
    xn2 = xbf0 * xbf0 + xbf1 * xbf1 + xbf2 * xbf2
    ybf0 = ybt[0:1, :].astype(jnp.float32)
    ybf1 = ybt[1:2, :].astype(jnp.float32)
    ybf2 = ybt[2:3, :].astype(jnp.float32)
    yn2 = ybf0 * ybf0 + ybf1 * ybf1 + ybf2 * ybf2

    c_row = yn2 + jnp.where(out_mask_r, 0.0, _BIG)
    a_col = xn2 + jnp.where(in_mask_c, 0.0, _BIG)

    a_hi = a_col.astype(jnp.bfloat16)
    a_lo = (a_col - a_hi.astype(jnp.float32)).astype(jnp.bfloat16)
    c_hi = c_row.astype(jnp.bfloat16)
    c_lo = (c_row - c_hi.astype(jnp.float32)).astype(jnp.bfloat16)

    ones_c = jnp.ones((_N, 1), jnp.bfloat16)
    ones_r = jnp.ones((1, _N), jnp.bfloat16)
    xp = jnp.concatenate(
        [xb * jnp.bfloat16(-2.0), ones_c, ones_c, a_hi, a_lo],
        axis=1)
    yp = jnp.concatenate(
        [ybt, c_hi, c_lo, ones_r, ones_r], axis=0)

    colmin = jnp.full((1, _N), _BIG, dtype=jnp.float32)
    row_mins = []
    for t in range(_N // _TILE):
        sl = slice(t * _TILE, (t + 1) * _TILE)
        m = jax.lax.dot_general(
            xp[sl, :], yp,
            dimension_numbers=(((1,), (0,)), ((), ())),
            preferred_element_type=jnp.float32)

        row_mins.append(jnp.min(m, axis=1, keepdims=True))
        colmin = jnp.minimum(colmin, jnp.min(m, axis=0, keepdims=True))

    row_min_all = jnp.maximum(jnp.concatenate(row_mins, axis=0), 0.0)
    rowsum = jnp.sum(jnp.where(in_mask_c, jnp.sqrt(row_min_all), 0.0))

    colmin = jnp.maximum(colmin, 0.0)
    sum_yx = jnp.sum(jnp.where(out_mask_r, jnp.sqrt(colmin), 0.0))

    n_in_part = jnp.maximum(1.0, n_in)
    n_out_part = jnp.maximum(1.0, n_out)

    chamfer = 0.5 * (rowsum / n_out_part + sum_yx / n_in_part)
    contrib = jnp.where(
        n_out == 0.0,
        x_norm_sum / n_in_part,
        jnp.where(n_in == 0.0, 0.0, chamfer),
    )
    acc_ref[0] = jnp.full((8, 128), 1.0 / _B, jnp.float32) * contrib


def _sc_rsqrt(sq):
    i = lax.bitcast_convert_type(sq, jnp.int32)
    i = jnp.int32(0x5F3759DF) - (i >> 1)
    g = lax.bitcast_convert_type(i, jnp.float32)
    for _ in range(3):
        g = g * (1.5 - 0.5 * sq * g * g)
    return g


def _ez_sc_body(yt_hbm, pid_hbm, out_hbm, y_v, pid_v, res_v, sem):
    wid = lax.axis_index("s") * 2 + lax.axis_index("c")

    @pl.when(wid < _B)
    def _():
        pltpu.async_copy(yt_hbm.at[wid], y_v, sem).wait()
        pltpu.async_copy(pid_hbm.at[wid], pid_v, sem).wait()

        def body(i, carry):
            s, c = carry
            sl = pl.ds(i * _SC_L, _SC_L)
            y0 = y_v[0, sl]
            y1 = y_v[1, sl]
            y2 = y_v[2, sl]
            sq = jnp.maximum(y0 * y0 + y1 * y1 + y2 * y2, 1e-35)
            norm = sq * _sc_rsqrt(sq)
            mask = pid_v[sl] == 0
            s = s + jnp.where(mask, norm, 0.0)
            c = c + jnp.where(mask, 1.0, 0.0)
            return s, c

        zero = jnp.zeros((_SC_L,), jnp.float32)
        s, c = lax.fori_loop(0, _N // _SC_L, body, (zero, zero))
        num = zero + jnp.sum(s)
        den = zero + jnp.maximum(1.0, jnp.sum(c))
        res_v[...] = num / den
        pltpu.sync_copy(res_v, out_hbm.at[wid])


_ez_sc_kernel = functools.partial(
    pl.kernel,
    mesh=plsc.VectorSubcoreMesh(core_axis_name="c", subcore_axis_name="s"),
    compiler_params=pltpu.CompilerParams(needs_layout_passes=False),
    out_type=jax.ShapeDtypeStruct((_B, _SC_L), jnp.float32),
    scratch_types=[
        pltpu.VMEM((_D, _N), jnp.float32),
        pltpu.VMEM((_N,), jnp.int32),
        pltpu.VMEM((_SC_L,), jnp.float32),
        pltpu.SemaphoreType.DMA,
    ],
)(_ez_sc_body)


@jax.jit
def kernel(target, reco, in_pid, out_pid):
    in_c = in_pid.astype(jnp.int32)[..., None]
    out_r = out_pid.astype(jnp.int32)[:, None, :]
    yt = jnp.transpose(reco, (0, 2, 1))

    ez_rows = _ez_sc_kernel(yt, out_pid.astype(jnp.int32))

    acc = pl.pallas_call(
        _chamfer_tc_body,
        grid=(_B,),
        in_specs=[
            pl.BlockSpec((1, _N, _D), lambda b: (b, 0, 0)),
            pl.BlockSpec((1, _D, _N), lambda b: (b, 0, 0)),
            pl.BlockSpec((1, _N, 1), lambda b: (b, 0, 0)),
            pl.BlockSpec((1, 1, _N), lambda b: (b, 0, 0)),
        ],
        out_specs=pl.BlockSpec((1, 8, 128), lambda b: (b, 0, 0)),
        out_shape=jax.ShapeDtypeStruct((_B, 8, 128), jnp.float32),
    )(target, yt, in_c, out_r)

    return jnp.sum(acc[:, 0, 0]), jnp.mean(ez_rows[:, 0])


# speedup vs baseline: 1.1916x; 1.1916x over previous
import jax
import jax.numpy as jnp
from jax import lax
from jax.experimental import pallas as pl

_B, _N, _D = 16, 2048, 3
_TILE = 256
_BIG = 1e30


def _chamfer_tc_body(x_ref, yt_ref, inp_ref, outp_ref, acc_ref):
    x = x_ref[0]
    in_pid = inp_ref[0]
    out_pid = outp_ref[0]

    in_mask_c = in_pid != 0
    out_mask_r = out_pid != 0

    n_in = jnp.sum(in_mask_c.astype(jnp.float32))
    n_out = jnp.sum(out_mask_r.astype(jnp.float32))

    x0c = x[:, 0:1]
    x1c = x[:, 1:2]
    x2c = x[:, 2:3]
    xn2f = x0c * x0c + x1c * x1c + x2c * x2c
    x_norm = jnp.sqrt(xn2f)
    x_norm_sum = jnp.sum(jnp.where(in_mask_c, x_norm, 0.0))

    xb = x.astype(jnp.bfloat16)
    ybt = yt_ref[0].astype(jnp.bfloat16)
    xbf0 = xb[:, 0:1].astype(jnp.float32)
    xbf1 = xb[:, 1:2].astype(jnp.float32)
    xbf2 = xb[:, 2:3].astype(jnp.float32)
    xn2 = xbf0 * xbf0 + xbf1 * xbf1 + xbf2 * xbf2
    ybf0 = ybt[0:1, :].astype(jnp.float32)
    ybf1 = ybt[1:2, :].astype(jnp.float32)
    ybf2 = ybt[2:3, :].astype(jnp.float32)
    yn2 = ybf0 * ybf0 + ybf1 * ybf1 + ybf2 * ybf2

    yt_f = yt_ref[0]
    y0r = yt_f[0:1, :]
    y1r = yt_f[1:2, :]
    y2r = yt_f[2:3, :]
    yn2f = y0r * y0r + y1r * y1r + y2r * y2r
    ez_mask = jnp.logical_not(out_mask_r)
    ez_sum = jnp.sum(jnp.where(ez_mask, jnp.sqrt(yn2f), 0.0))
    ez_cnt = jnp.maximum(1.0, jnp.sum(ez_mask.astype(jnp.float32)))
    ez = ez_sum / ez_cnt

    c_row = yn2 + jnp.where(out_mask_r, 0.0, _BIG)
    a_col = xn2 + jnp.where(in_mask_c, 0.0, _BIG)

    a_hi = a_col.astype(jnp.bfloat16)
    a_lo = (a_col - a_hi.astype(jnp.float32)).astype(jnp.bfloat16)
    c_hi = c_row.astype(jnp.bfloat16)
    c_lo = (c_row - c_hi.astype(jnp.float32)).astype(jnp.bfloat16)

    ones_c = jnp.ones((_N, 1), jnp.bfloat16)
    ones_r = jnp.ones((1, _N), jnp.bfloat16)
    xp = jnp.concatenate(
        [xb * jnp.bfloat16(-2.0), ones_c, ones_c, a_hi, a_lo],
        axis=1)
    yp = jnp.concatenate(
        [ybt, c_hi, c_lo, ones_r, ones_r], axis=0)

    colmin = jnp.full((1, _N), _BIG, dtype=jnp.float32)
    row_mins = []
    for t in range(_N // _TILE):
        sl = slice(t * _TILE, (t + 1) * _TILE)
        m = jax.lax.dot_general(
            xp[sl, :], yp,
            dimension_numbers=(((1,), (0,)), ((), ())),
            preferred_element_type=jnp.float32)

        row_mins.append(jnp.min(m, axis=1, keepdims=True))
        colmin = jnp.minimum(colmin, jnp.min(m, axis=0, keepdims=True))

    row_min_all = jnp.maximum(jnp.concatenate(row_mins, axis=0), 0.0)
    rowsum = jnp.sum(jnp.where(in_mask_c, jnp.sqrt(row_min_all), 0.0))

    colmin = jnp.maximum(colmin, 0.0)
    sum_yx = jnp.sum(jnp.where(out_mask_r, jnp.sqrt(colmin), 0.0))

    n_in_part = jnp.maximum(1.0, n_in)
    n_out_part = jnp.maximum(1.0, n_out)

    chamfer = 0.5 * (rowsum / n_out_part + sum_yx / n_in_part)
    contrib = jnp.where(
        n_out == 0.0,
        x_norm_sum / n_in_part,
        jnp.where(n_in == 0.0, 0.0, chamfer),
    )
    row_id = lax.broadcasted_iota(jnp.int32, (8, 128), 0)
    vals = jnp.where(row_id == 0, contrib, jnp.where(row_id == 1, ez, 0.0))
    acc_ref[0] = vals * (1.0 / _B)


@jax.jit
def kernel(target, reco, in_pid, out_pid):
    in_c = in_pid.astype(jnp.int32)[..., None]
    out_r = out_pid.astype(jnp.int32)[:, None, :]
    yt = jnp.transpose(reco, (0, 2, 1))

    acc = pl.pallas_call(
        _chamfer_tc_body,
        grid=(_B,),
        in_specs=[
            pl.BlockSpec((1, _N, _D), lambda b: (b, 0, 0)),
            pl.BlockSpec((1, _D, _N), lambda b: (b, 0, 0)),
            pl.BlockSpec((1, _N, 1), lambda b: (b, 0, 0)),
            pl.BlockSpec((1, 1, _N), lambda b: (b, 0, 0)),
        ],
        out_specs=pl.BlockSpec((1, 8, 128), lambda b: (b, 0, 0)),
        out_shape=jax.ShapeDtypeStruct((_B, 8, 128), jnp.float32),
    )(target, yt, in_c, out_r)

    return jnp.sum(acc[:, 0, 0]), jnp.sum(acc[:, 1, 0])


# speedup vs baseline: 1.2616x; 1.0588x over previous
import jax
import jax.numpy as jnp
from jax import lax
from jax.experimental import pallas as pl

_B, _N, _D = 16, 2048, 3
_TILE = 256
_BIG = 1e30
_BPG = 2


def _chamfer_tc_body(x_ref, yt_ref, inp_ref, outp_ref, acc_ref):
    for j in range(_BPG):
        acc_ref[j] = _one_batch(x_ref[j], yt_ref[j], inp_ref[j], outp_ref[j])


def _one_batch(x, yt_b, in_pid, out_pid):

    in_mask_c = in_pid != 0
    out_mask_r = out_pid != 0

    n_in = jnp.sum(in_mask_c.astype(jnp.float32))
    n_out = jnp.sum(out_mask_r.astype(jnp.float32))

    x0c = x[:, 0:1]
    x1c = x[:, 1:2]
    x2c = x[:, 2:3]
    xn2f = x0c * x0c + x1c * x1c + x2c * x2c
    x_norm = jnp.sqrt(xn2f)
    x_norm_sum = jnp.sum(jnp.where(in_mask_c, x_norm, 0.0))

    xb = x.astype(jnp.bfloat16)
    ybt = yt_b.astype(jnp.bfloat16)
    xbf0 = xb[:, 0:1].astype(jnp.float32)
    xbf1 = xb[:, 1:2].astype(jnp.float32)
    xbf2 = xb[:, 2:3].astype(jnp.float32)
    xn2 = xbf0 * xbf0 + xbf1 * xbf1 + xbf2 * xbf2
    ybf0 = ybt[0:1, :].astype(jnp.float32)
    ybf1 = ybt[1:2, :].astype(jnp.float32)
    ybf2 = ybt[2:3, :].astype(jnp.float32)
    yn2 = ybf0 * ybf0 + ybf1 * ybf1 + ybf2 * ybf2

    y0r = yt_b[0:1, :]
    y1r = yt_b[1:2, :]
    y2r = yt_b[2:3, :]
    yn2f = y0r * y0r + y1r * y1r + y2r * y2r
    ez_mask = jnp.logical_not(out_mask_r)
    ez_sum = jnp.sum(jnp.where(ez_mask, jnp.sqrt(yn2f), 0.0))
    ez_cnt = jnp.maximum(1.0, jnp.sum(ez_mask.astype(jnp.float32)))
    ez = ez_sum / ez_cnt

    c_row = yn2 + jnp.where(out_mask_r, 0.0, _BIG)
    a_col = xn2 + jnp.where(in_mask_c, 0.0, _BIG)

    a_hi = a_col.astype(jnp.bfloat16)
    a_lo = (a_col - a_hi.astype(jnp.float32)).astype(jnp.bfloat16)
    c_hi = c_row.astype(jnp.bfloat16)
    c_lo = (c_row - c_hi.astype(jnp.float32)).astype(jnp.bfloat16)

    ones_c = jnp.ones((_N, 1), jnp.bfloat16)
    ones_r = jnp.ones((1, _N), jnp.bfloat16)
    xp = jnp.concatenate(
        [xb * jnp.bfloat16(-2.0), ones_c, ones_c, a_hi, a_lo],
        axis=1)
    yp = jnp.concatenate(
        [ybt, c_hi, c_lo, ones_r, ones_r], axis=0)

    colmin = jnp.full((1, _N), _BIG, dtype=jnp.float32)
    row_mins = []
    for t in range(_N // _TILE):
        sl = slice(t * _TILE, (t + 1) * _TILE)
        m = jax.lax.dot_general(
            xp[sl, :], yp,
            dimension_numbers=(((1,), (0,)), ((), ())),
            preferred_element_type=jnp.float32)

        row_mins.append(jnp.min(m, axis=1, keepdims=True))
        colmin = jnp.minimum(colmin, jnp.min(m, axis=0, keepdims=True))

    row_min_all = jnp.maximum(jnp.concatenate(row_mins, axis=0), 0.0)
    rowsum = jnp.sum(jnp.where(in_mask_c, jnp.sqrt(row_min_all), 0.0))

    colmin = jnp.maximum(colmin, 0.0)
    sum_yx = jnp.sum(jnp.where(out_mask_r, jnp.sqrt(colmin), 0.0))

    n_in_part = jnp.maximum(1.0, n_in)
    n_out_part = jnp.maximum(1.0, n_out)

    chamfer = 0.5 * (rowsum / n_out_part + sum_yx / n_in_part)
    contrib = jnp.where(
        n_out == 0.0,
        x_norm_sum / n_in_part,
        jnp.where(n_in == 0.0, 0.0, chamfer),
    )
    row_id = lax.broadcasted_iota(jnp.int32, (8, 128), 0)
    vals = jnp.where(row_id == 0, contrib, jnp.where(row_id == 1, ez, 0.0))
    return vals * (1.0 / _B)


@jax.jit
def kernel(target, reco, in_pid, out_pid):
    in_c = in_pid.astype(jnp.int32)[..., None]
    out_r = out_pid.astype(jnp.int32)[:, None, :]
    yt = jnp.transpose(reco, (0, 2, 1))

    acc = pl.pallas_call(
        _chamfer_tc_body,
        grid=(_B // _BPG,),
        in_specs=[
            pl.BlockSpec((_BPG, _N, _D), lambda b: (b, 0, 0)),
            pl.BlockSpec((_BPG, _D, _N), lambda b: (b, 0, 0)),
            pl.BlockSpec((_BPG, _N, 1), lambda b: (b, 0, 0)),
            pl.BlockSpec((_BPG, 1, _N), lambda b: (b, 0, 0)),
        ],
        out_specs=pl.BlockSpec((_BPG, 8, 128), lambda b: (b, 0, 0)),
        out_shape=jax.ShapeDtypeStruct((_B, 8, 128), jnp.float32),
    )(target, yt, in_c, out_r)

    return jnp.sum(acc[:, 0, 0]), jnp.sum(acc[:, 1, 0])
